# baseline (speedup 1.0000x reference)
import jax
import jax.numpy as jnp
from jax.experimental import pallas as pl
from jax.experimental.pallas import tpu as pltpu

Q = 1024
K = 100000
D = 64
BK = 1000
NB = K // BK
NG = BK // 8


def _nn_kernel(x_ref, idx_ref, xsq_ref, dist_out, idx_out,
               xm2_ref, minv_ref, grp_ref):
    j = pl.program_id(0)

    @pl.when(j == 0)
    def _init():
        xm2_ref[...] = x_ref[...] * jnp.float32(-2.0)
        minv_ref[...] = jnp.full((8, Q), jnp.inf, jnp.float32)
        grp_ref[...] = jnp.zeros((8, Q), jnp.int32)

    blk = idx_ref[...]
    m2 = jax.lax.dot_general(
        blk, xm2_ref[...], (((1,), (1,)), ((), ())),
        preferred_element_type=jnp.float32)

    isq = jnp.sum(blk * blk, axis=1, keepdims=True)
    t = (xsq_ref[...] + isq) + m2

    minv8 = t[0:8, :]
    gbest = jnp.zeros((8, Q), jnp.int32)
    for g in range(1, NG):
        tg = t[g * 8:(g + 1) * 8, :]
        better = tg < minv8
        minv8 = jnp.where(better, tg, minv8)
        gbest = jnp.where(better, g, gbest)

    better = minv8 < minv_ref[...]
    minv_ref[...] = jnp.where(better, minv8, minv_ref[...])
    grp_ref[...] = jnp.where(better, j * NG + gbest, grp_ref[...])

    @pl.when(j == NB - 1)
    def _emit():
        minv = minv_ref[...]
        srow = jax.lax.broadcasted_iota(jnp.int32, (8, Q), 0)
        rid = grp_ref[...] * 8 + srow
        gmin = jnp.min(minv, axis=0, keepdims=True)
        cand = jnp.where(minv == gmin, rid, K)
        gidx = jnp.min(cand, axis=0, keepdims=True)
        dist_out[...] = jnp.maximum(gmin, 0.0).reshape(Q, 1)
        idx_out[...] = gidx.reshape(Q, 1)


@jax.jit
def kernel(x, index):
    x_sq = jnp.sum(x * x, axis=1)[None, :]
    dist, idx = pl.pallas_call(
        _nn_kernel,
        grid=(NB,),
        in_specs=[
            pl.BlockSpec((Q, D), lambda j: (0, 0)),
            pl.BlockSpec((BK, D), lambda j: (j, 0)),
            pl.BlockSpec((1, Q), lambda j: (0, 0)),
        ],
        out_specs=[
            pl.BlockSpec((Q, 1), lambda j: (0, 0)),
            pl.BlockSpec((Q, 1), lambda j: (0, 0)),
        ],
        out_shape=[
            jax.ShapeDtypeStruct((Q, 1), jnp.float32),
            jax.ShapeDtypeStruct((Q, 1), jnp.int32),
        ],
        scratch_shapes=[
            pltpu.VMEM((Q, D), jnp.float32),
            pltpu.VMEM((8, Q), jnp.float32),
            pltpu.VMEM((8, Q), jnp.int32),
        ],
        compiler_params=pltpu.CompilerParams(
            dimension_semantics=("arbitrary",),
        ),
    )(x, index, x_sq)
    return dist, idx


# speedup vs baseline: 1.0148x; 1.0148x over previous
import jax
import jax.numpy as jnp
from jax.experimental import pallas as pl
from jax.experimental.pallas import tpu as pltpu

Q = 1024
K = 100000
D = 64
BK = 1000
NB = K // BK
NG = BK // 8


def _nn_kernel(xm2_ref, idx_ref, xsq_ref, minv_ref, grp_ref):
    j = pl.program_id(0)

    blk = idx_ref[...]
    m2 = jax.lax.dot_general(
        blk, xm2_ref[...], (((1,), (1,)), ((), ())),
        preferred_element_type=jnp.float32)

    isq = jnp.sum(blk * blk, axis=1, keepdims=True)
    t = (xsq_ref[...] + isq) + m2

    minv8 = t[0:8, :]
    gbest = jnp.zeros((8, Q), jnp.int32)
    for g in range(1, NG):
        tg = t[g * 8:(g + 1) * 8, :]
        better = tg < minv8
        minv8 = jnp.where(better, tg, minv8)
        gbest = jnp.where(better, g, gbest)

    better = (minv8 < minv_ref[...]) | (j == 0)
    minv_ref[...] = jnp.where(better, minv8, minv_ref[...])
    grp_ref[...] = jnp.where(better, j * NG + gbest, grp_ref[...])


@jax.jit
def kernel(x, index):
    x_sq = jnp.sum(x * x, axis=1)[None, :]
    x_m2 = x * jnp.float32(-2.0)
    minv, grp = pl.pallas_call(
        _nn_kernel,
        grid=(NB,),
        in_specs=[
            pl.BlockSpec((Q, D), lambda j: (0, 0)),
            pl.BlockSpec((BK, D), lambda j: (j, 0)),
            pl.BlockSpec((1, Q), lambda j: (0, 0)),
        ],
        out_specs=[
            pl.BlockSpec((8, Q), lambda j: (0, 0)),
            pl.BlockSpec((8, Q), lambda j: (0, 0)),
        ],
        out_shape=[
            jax.ShapeDtypeStruct((8, Q), jnp.float32),
            jax.ShapeDtypeStruct((8, Q), jnp.int32),
        ],
        compiler_params=pltpu.CompilerParams(
            dimension_semantics=("arbitrary",),
        ),
    )(x_m2, index, x_sq)

    rid = grp * 8 + jnp.arange(8, dtype=jnp.int32)[:, None]
    gmin = jnp.min(minv, axis=0)
    gidx = jnp.min(jnp.where(minv == gmin[None, :], rid, K), axis=0)
    return jnp.maximum(gmin, 0.0)[:, None], gidx[:, None]


# speedup vs baseline: 1.1506x; 1.1338x over previous
import jax
import jax.numpy as jnp
from jax.experimental import pallas as pl
from jax.experimental.pallas import tpu as pltpu

Q = 1024
K = 100000
D = 64
BK = 2000
NB = K // BK
NG = BK // 8


def _nn_kernel(xm2_ref, idx_ref, xsq_ref, minv_ref, grp_ref):
    j = pl.program_id(0)

    blk = idx_ref[...]
    m2 = jax.lax.dot_general(
        blk, xm2_ref[...], (((1,), (1,)), ((), ())),
        preferred_element_type=jnp.float32)

    isq = jnp.sum(blk * blk, axis=1, keepdims=True)
    t = (xsq_ref[...] + isq) + m2

    minv8 = t[0:8, :]
    gbest = jnp.zeros((8, Q), jnp.int32)
    for g in range(1, NG):
        tg = t[g * 8:(g + 1) * 8, :]
        better = tg < minv8
        minv8 = jnp.where(better, tg, minv8)
        gbest = jnp.where(better, g, gbest)

    better = (minv8 < minv_ref[...]) | (j == 0)
    minv_ref[...] = jnp.where(better, minv8, minv_ref[...])
    grp_ref[...] = jnp.where(better, j * NG + gbest, grp_ref[...])


@jax.jit
def kernel(x, index):
    x_sq = jnp.sum(x * x, axis=1)[None, :]
    x_m2 = x * jnp.float32(-2.0)
    minv, grp = pl.pallas_call(
        _nn_kernel,
        grid=(NB,),
        in_specs=[
            pl.BlockSpec((Q, D), lambda j: (0, 0)),
            pl.BlockSpec((BK, D), lambda j: (j, 0)),
            pl.BlockSpec((1, Q), lambda j: (0, 0)),
        ],
        out_specs=[
            pl.BlockSpec((8, Q), lambda j: (0, 0)),
            pl.BlockSpec((8, Q), lambda j: (0, 0)),
        ],
        out_shape=[
            jax.ShapeDtypeStruct((8, Q), jnp.float32),
            jax.ShapeDtypeStruct((8, Q), jnp.int32),
        ],
        compiler_params=pltpu.CompilerParams(
            dimension_semantics=("arbitrary",),
        ),
    )(x_m2, index, x_sq)

    rid = grp * 8 + jnp.arange(8, dtype=jnp.int32)[:, None]
    gmin = jnp.min(minv, axis=0)
    gidx = jnp.min(jnp.where(minv == gmin[None, :], rid, K), axis=0)
    return jnp.maximum(gmin, 0.0)[:, None], gidx[:, None]


# speedup vs baseline: 1.1958x; 1.0393x over previous
import jax
import jax.numpy as jnp
from jax.experimental import pallas as pl
from jax.experimental.pallas import tpu as pltpu

Q = 1024
K = 100000
D = 64
BK = 4000
NB = K // BK
NG = BK // 8


def _nn_kernel(xm2_ref, idx_ref, xsq_ref, minv_ref, grp_ref):
    j = pl.program_id(0)

    blk = idx_ref[...]
    m2 = jax.lax.dot_general(
        blk, xm2_ref[...], (((1,), (1,)), ((), ())),
        preferred_element_type=jnp.float32)

    isq = jnp.sum(blk * blk, axis=1, keepdims=True)
    t = (xsq_ref[...] + isq) + m2

    minv8 = t[0:8, :]
    gbest = jnp.zeros((8, Q), jnp.int32)
    for g in range(1, NG):
        tg = t[g * 8:(g + 1) * 8, :]
        better = tg < minv8
        minv8 = jnp.where(better, tg, minv8)
        gbest = jnp.where(better, g, gbest)

    better = (minv8 < minv_ref[...]) | (j == 0)
    minv_ref[...] = jnp.where(better, minv8, minv_ref[...])
    grp_ref[...] = jnp.where(better, j * NG + gbest, grp_ref[...])


@jax.jit
def kernel(x, index):
    x_sq = jnp.sum(x * x, axis=1)[None, :]
    x_m2 = x * jnp.float32(-2.0)
    minv, grp = pl.pallas_call(
        _nn_kernel,
        grid=(NB,),
        in_specs=[
            pl.BlockSpec((Q, D), lambda j: (0, 0)),
            pl.BlockSpec((BK, D), lambda j: (j, 0)),
            pl.BlockSpec((1, Q), lambda j: (0, 0)),
        ],
        out_specs=[
            pl.BlockSpec((8, Q), lambda j: (0, 0)),
            pl.BlockSpec((8, Q), lambda j: (0, 0)),
        ],
        out_shape=[
            jax.ShapeDtypeStruct((8, Q), jnp.float32),
            jax.ShapeDtypeStruct((8, Q), jnp.int32),
        ],
        compiler_params=pltpu.CompilerParams(
            dimension_semantics=("arbitrary",),
        ),
    )(x_m2, index, x_sq)

    rid = grp * 8 + jnp.arange(8, dtype=jnp.int32)[:, None]
    gmin = jnp.min(minv, axis=0)
    gidx = jnp.min(jnp.where(minv == gmin[None, :], rid, K), axis=0)
    return jnp.maximum(gmin, 0.0)[:, None], gidx[:, None]


# speedup vs baseline: 1.2195x; 1.0198x over previous
import jax
import jax.numpy as jnp
from jax.experimental import pallas as pl
from jax.experimental.pallas import tpu as pltpu

Q = 1024
K = 100000
D = 64
BK = 10000
NB = K // BK
NG = BK // 8


def _nn_kernel(xm2_ref, idx_ref, xsq_ref, minv_ref, grp_ref):
    j = pl.program_id(0)

    blk = idx_ref[...]
    m2 = jax.lax.dot_general(
        blk, xm2_ref[...], (((1,), (1,)), ((), ())),
        preferred_element_type=jnp.float32)

    isq = jnp.sum(blk * blk, axis=1, keepdims=True)
    t = (xsq_ref[...] + isq) + m2

    minv8 = t[0:8, :]
    gbest = jnp.zeros((8, Q), jnp.int32)
    for g in range(1, NG):
        tg = t[g * 8:(g + 1) * 8, :]
        better = tg < minv8
        minv8 = jnp.where(better, tg, minv8)
        gbest = jnp.where(better, g, gbest)

    better = (minv8 < minv_ref[...]) | (j == 0)
    minv_ref[...] = jnp.where(better, minv8, minv_ref[...])
    grp_ref[...] = jnp.where(better, j * NG + gbest, grp_ref[...])


@jax.jit
def kernel(x, index):
    x_sq = jnp.sum(x * x, axis=1)[None, :]
    x_m2 = x * jnp.float32(-2.0)
    minv, grp = pl.pallas_call(
        _nn_kernel,
        grid=(NB,),
        in_specs=[
            pl.BlockSpec((Q, D), lambda j: (0, 0)),
            pl.BlockSpec((BK, D), lambda j: (j, 0)),
            pl.BlockSpec((1, Q), lambda j: (0, 0)),
        ],
        out_specs=[
            pl.BlockSpec((8, Q), lambda j: (0, 0)),
            pl.BlockSpec((8, Q), lambda j: (0, 0)),
        ],
        out_shape=[
            jax.ShapeDtypeStruct((8, Q), jnp.float32),
            jax.ShapeDtypeStruct((8, Q), jnp.int32),
        ],
        compiler_params=pltpu.CompilerParams(
            dimension_semantics=("arbitrary",),
        ),
    )(x_m2, index, x_sq)

    rid = grp * 8 + jnp.arange(8, dtype=jnp.int32)[:, None]
    gmin = jnp.min(minv, axis=0)
    gidx = jnp.min(jnp.where(minv == gmin[None, :], rid, K), axis=0)
    return jnp.maximum(gmin, 0.0)[:, None], gidx[:, None]
